# baseline (speedup 1.0000x reference)
import dataclasses

import jax
import jax.numpy as jnp
from jax import lax
from jax.experimental import pallas as pl
from jax.experimental.pallas import tpu as pltpu
from jax.experimental.pallas import tpu_sc as plsc

DIM = 64
PAIR_DIM = 128
WINDOW = 128
LANES = 16
NC = 2
NS = 16
NW = NC * NS


def _gather_kernel(b, s):
    num_indices = b * s
    chunks = num_indices // WINDOW
    cpw = chunks // NW
    cps = b // WINDOW
    mesh = plsc.VectorSubcoreMesh(core_axis_name="c", subcore_axis_name="s")
    cp = pltpu.CompilerParams()
    if "needs_layout_passes" in pltpu.CompilerParams.__dataclass_fields__:
        cp = dataclasses.replace(cp, needs_layout_passes=False)

    @pl.kernel(
        out_type=jax.ShapeDtypeStruct((s, DIM, b), jnp.float32),
        mesh=mesh,
        compiler_params=cp,
        scratch_types=[
            pltpu.VMEM((cpw, WINDOW), jnp.int32),
            pltpu.VMEM((cpw, WINDOW), jnp.int32),
            pltpu.VMEM((WINDOW, PAIR_DIM), jnp.float32),
            pltpu.VMEM((DIM, WINDOW), jnp.float32),
            pltpu.SemaphoreType.DMA,
        ],
    )
    def kern(table_hbm, idx_hbm, par_hbm, out_hbm, idx_v, par_v, rows_v, t_v, sem):
        wid = lax.axis_index("s") * NC + lax.axis_index("c")
        pltpu.sync_copy(idx_hbm.at[wid], idx_v)
        pltpu.sync_copy(par_hbm.at[wid], par_v)

        @pl.loop(0, cpw)
        def _(j):
            pltpu.async_copy(table_hbm.at[idx_v.at[j]], rows_v, sem).wait()

            iota = lax.iota(jnp.int32, LANES)
            for g in range(WINDOW // LANES):
                row = iota + (g * LANES)
                par = par_v.at[j][pl.ds(g * LANES, LANES)]
                for d in range(DIM):
                    t_v.at[d][pl.ds(g * LANES, LANES)] = plsc.load_gather(
                        rows_v, [row, par + d]
                    )

            r = wid * cpw + j
            si = r // cps
            b0 = (r % cps) * WINDOW
            pltpu.sync_copy(t_v, out_hbm.at[si].at[:, pl.ds(b0, WINDOW)])

    return kern


def kernel(x, weight):
    b, s = x.shape
    n = b * s
    xt = x.T.astype(jnp.int32)
    cpw = n // (NW * WINDOW)
    idx = (xt >> 1).reshape(NW, cpw, WINDOW)
    par = ((xt & 1) << 6).reshape(NW, cpw, WINDOW)
    table = weight.reshape(weight.shape[0] // 2, PAIR_DIM)
    out_t = _gather_kernel(b, s)(table, idx, par)
    return jnp.transpose(out_t, (2, 0, 1))


# speedup vs baseline: 1.8079x; 1.8079x over previous
import jax
import jax.numpy as jnp
from jax import lax
from jax.experimental import pallas as pl
from jax.experimental.pallas import tpu as pltpu
from jax.experimental.pallas import tpu_sc as plsc

DIM = 64
PAIR_DIM = 128
WINDOW = 128
NC = 2
NS = 16
NW = NC * NS
PHASES = 2


def _gather_kernel(num_indices):
    chunks = num_indices // WINDOW
    cpw = chunks // NW
    mesh = plsc.VectorSubcoreMesh(core_axis_name="c", subcore_axis_name="s")

    @pl.kernel(
        out_type=jax.ShapeDtypeStruct((num_indices, PAIR_DIM), jnp.float32),
        mesh=mesh,
        scratch_types=[
            pltpu.VMEM((cpw, WINDOW), jnp.int32),
            pltpu.VMEM((WINDOW, PAIR_DIM), jnp.float32),
            pltpu.SemaphoreType.DMA,
        ],
    )
    def kern(table_hbm, idx_hbm, out_hbm, idx_v, rows_v, sem):
        wid = lax.axis_index("s") * NC + lax.axis_index("c")
        pltpu.sync_copy(idx_hbm.at[wid], idx_v)

        @pl.loop(0, cpw)
        def _(j):
            pltpu.async_copy(table_hbm.at[idx_v.at[j]], rows_v, sem).wait()
            base = (wid * cpw + j) * WINDOW
            pltpu.sync_copy(rows_v, out_hbm.at[pl.ds(base, WINDOW)])

    return kern


def _tail_first_kernel(res_ref, xt_ref, o_ref):
    data = res_ref[...]
    par = (xt_ref[0, 0] & 1)[:, None] == 1
    sel = jnp.where(par, data[:, DIM:], data[:, :DIM])
    o_ref[0] = sel.T


def _tail_next_kernel(prev_ref, res_ref, xt_ref, o_ref):
    del prev_ref
    _tail_first_kernel(res_ref, xt_ref, o_ref)


def _tail(res, xt, prev, b, s, sp, s0):
    out_shape = jax.ShapeDtypeStruct((s, DIM, b), jnp.float32)
    res_spec = pl.BlockSpec((b, PAIR_DIM), lambda i: (i, 0))
    xt_spec = pl.BlockSpec((1, 1, b), lambda i: (i, 0, 0))
    out_spec = pl.BlockSpec((1, DIM, b), lambda i: (i + s0, 0, 0))
    if prev is None:
        return pl.pallas_call(
            _tail_first_kernel,
            grid=(sp,),
            in_specs=[res_spec, xt_spec],
            out_specs=out_spec,
            out_shape=out_shape,
        )(res, xt)
    return pl.pallas_call(
        _tail_next_kernel,
        grid=(sp,),
        in_specs=[pl.BlockSpec(memory_space=pltpu.MemorySpace.HBM), res_spec, xt_spec],
        out_specs=out_spec,
        out_shape=out_shape,
        input_output_aliases={0: 0},
    )(prev, res, xt)


def kernel(x, weight):
    b, s = x.shape
    table = weight.reshape(weight.shape[0] // 2, PAIR_DIM)
    sp = s // PHASES
    np_idx = b * sp
    gather = _gather_kernel(np_idx)
    xts = []
    ress = []
    for k in range(PHASES):
        xt_k = x[:, k * sp:(k + 1) * sp].T.astype(jnp.int32)
        idx_k = (xt_k >> 1).reshape(NW, np_idx // (NW * WINDOW), WINDOW)
        xts.append(xt_k)
        ress.append(gather(table, idx_k))
    out = None
    for k in range(PHASES):
        out = _tail(ress[k], xts[k].reshape(sp, 1, b), out, b, s, sp, k * sp)
    return jnp.transpose(out, (2, 0, 1))


# speedup vs baseline: 1.8734x; 1.0362x over previous
import jax
import jax.numpy as jnp
from jax import lax
from jax.experimental import pallas as pl
from jax.experimental.pallas import tpu as pltpu
from jax.experimental.pallas import tpu_sc as plsc

DIM = 64
PAIR_DIM = 128
WINDOW = 128
NC = 2
NS = 16
NW = NC * NS
PHASES = 5


def _gather_kernel(num_indices):
    chunks = num_indices // WINDOW
    cpw = chunks // NW
    mesh = plsc.VectorSubcoreMesh(core_axis_name="c", subcore_axis_name="s")

    @pl.kernel(
        out_type=jax.ShapeDtypeStruct((num_indices, PAIR_DIM), jnp.float32),
        mesh=mesh,
        scratch_types=[
            pltpu.VMEM((cpw, WINDOW), jnp.int32),
            pltpu.VMEM((WINDOW, PAIR_DIM), jnp.float32),
            pltpu.SemaphoreType.DMA,
        ],
    )
    def kern(table_hbm, idx_hbm, out_hbm, idx_v, rows_v, sem):
        wid = lax.axis_index("s") * NC + lax.axis_index("c")
        pltpu.sync_copy(idx_hbm.at[wid], idx_v)

        @pl.loop(0, cpw)
        def _(j):
            pltpu.async_copy(table_hbm.at[idx_v.at[j]], rows_v, sem).wait()
            base = (wid * cpw + j) * WINDOW
            pltpu.sync_copy(rows_v, out_hbm.at[pl.ds(base, WINDOW)])

    return kern


def _tail_first_kernel(res_ref, xt_ref, o_ref):
    data = res_ref[...]
    par = (xt_ref[0, 0] & 1)[:, None] == 1
    sel = jnp.where(par, data[:, DIM:], data[:, :DIM])
    o_ref[0] = sel.T


def _tail_next_kernel(prev_ref, res_ref, xt_ref, o_ref):
    del prev_ref
    _tail_first_kernel(res_ref, xt_ref, o_ref)


def _tail(res, xt, prev, b, s, sp, s0):
    out_shape = jax.ShapeDtypeStruct((s, DIM, b), jnp.float32)
    res_spec = pl.BlockSpec((b, PAIR_DIM), lambda i: (i, 0))
    xt_spec = pl.BlockSpec((1, 1, b), lambda i: (i, 0, 0))
    out_spec = pl.BlockSpec((1, DIM, b), lambda i: (i + s0, 0, 0))
    if prev is None:
        return pl.pallas_call(
            _tail_first_kernel,
            grid=(sp,),
            in_specs=[res_spec, xt_spec],
            out_specs=out_spec,
            out_shape=out_shape,
        )(res, xt)
    return pl.pallas_call(
        _tail_next_kernel,
        grid=(sp,),
        in_specs=[pl.BlockSpec(memory_space=pltpu.MemorySpace.HBM), res_spec, xt_spec],
        out_specs=out_spec,
        out_shape=out_shape,
        input_output_aliases={0: 0},
    )(prev, res, xt)


def kernel(x, weight):
    b, s = x.shape
    table = weight.reshape(weight.shape[0] // 2, PAIR_DIM)
    sp = s // PHASES
    np_idx = b * sp
    gather = _gather_kernel(np_idx)
    xts = []
    ress = []
    for k in range(PHASES):
        xt_k = x[:, k * sp:(k + 1) * sp].T.astype(jnp.int32)
        idx_k = (xt_k >> 1).reshape(NW, np_idx // (NW * WINDOW), WINDOW)
        xts.append(xt_k)
        ress.append(gather(table, idx_k))
    out = None
    for k in range(PHASES):
        out = _tail(ress[k], xts[k].reshape(sp, 1, b), out, b, s, sp, k * sp)
    return jnp.transpose(out, (2, 0, 1))
